# baseline (speedup 1.0000x reference)
import functools

import jax
import jax.numpy as jnp
from jax.experimental import pallas as pl
from jax.experimental.pallas import tpu as pltpu


def _combine_body(s_ref, w_ref, c_ref):
    c32 = jax.lax.dot_general(
        s_ref[:], w_ref[:], (((1,), (1,)), ((), ())),
        preferred_element_type=jnp.float32)
    c_ref[:] = c32.astype(jnp.bfloat16)


def _stream_body(x_ref, c_ref, b_ref, o_ref):
    x_bf = x_ref[:].astype(jnp.bfloat16)
    acc = jnp.dot(x_bf, c_ref[:], preferred_element_type=jnp.float32)
    o_ref[:] = jnp.maximum(acc + b_ref[:], 0.0)


@functools.partial(jax.jit, static_argnames=())
def kernel(x, support, W, b):
    n, d = x.shape
    out_c, in_c = W.shape

    c = pl.pallas_call(
        _combine_body,
        out_shape=jax.ShapeDtypeStruct((d, out_c), jnp.bfloat16),
    )(support, W)

    bn = 1000
    out = pl.pallas_call(
        _stream_body,
        grid=(n // bn,),
        in_specs=[
            pl.BlockSpec((bn, d), lambda i: (i, 0)),
            pl.BlockSpec((d, out_c), lambda i: (0, 0)),
            pl.BlockSpec((1, out_c), lambda i: (0, 0)),
        ],
        out_specs=pl.BlockSpec((bn, out_c), lambda i: (i, 0)),
        out_shape=jax.ShapeDtypeStruct((n, out_c), jnp.float32),
        compiler_params=pltpu.CompilerParams(
            dimension_semantics=("parallel",)),
    )(x, c, b.reshape(1, out_c))
    return out
